# baseline (speedup 1.0000x reference)
import functools

import jax
import jax.numpy as jnp
from jax import lax
from jax.experimental import pallas as pl
from jax.experimental.pallas import tpu as pltpu
from jax.experimental.pallas import tpu_sc as plsc

N = 10000
B = 8
NP = N // B
NPAD = 1280
NT = B * NPAD
K = 16
TILE = 256
TPG = NPAD // TILE
GRID3 = NT // TILE
RT = 256
E = NT * K
NWORK = 32
EPW = E // NWORK
CH = 128
NCH = EPW // CH

_PREC = lax.Precision.DEFAULT


def _dot(a, b):
    return lax.dot_general(a, b, (((1,), (0,)), ((), ())),
                           precision=_PREC, preferred_element_type=jnp.float32)


def _knn_body(rows_ref, cols_ref, idx_ref):
    g = pl.program_id(0)
    rows = rows_ref[0]
    cols = cols_ref[0]
    d = jnp.zeros((RT, NPAD), jnp.float32)
    for c in range(4):
        df = rows[:, c:c + 1] - cols[c:c + 1, :]
        d = d + df * df
    colid = lax.broadcasted_iota(jnp.int32, (RT, NPAD), 1)
    picks = []
    for _ in range(K):
        m = jnp.min(d, axis=1, keepdims=True)
        cand = jnp.where(d == m, colid, jnp.int32(2**30))
        j = jnp.min(cand, axis=1, keepdims=True)
        picks.append(j)
        d = jnp.where(colid == j, jnp.float32(jnp.inf), d)
    idx_ref[0] = jnp.concatenate(picks, axis=1) + g * NPAD


def _knn(posg, ptr):
    return pl.pallas_call(
        _knn_body,
        grid=(B, NPAD // RT),
        in_specs=[
            pl.BlockSpec((1, RT, 4), lambda g, r: (g, r, 0)),
            pl.BlockSpec((1, 4, NPAD), lambda g, r: (g, 0, 0)),
        ],
        out_specs=pl.BlockSpec((1, RT, K), lambda g, r: (g, r, 0)),
        out_shape=jax.ShapeDtypeStruct((B, NPAD, K), jnp.int32),
    )(posg, ptr)


def _gather_rows(table, idxf):
    mesh = plsc.VectorSubcoreMesh(core_axis_name="c", subcore_axis_name="s")

    @functools.partial(
        pl.kernel, mesh=mesh,
        compiler_params=pltpu.CompilerParams(use_tc_tiling_on_sc=False),
        out_type=jax.ShapeDtypeStruct((E, 16), jnp.float32),
        scratch_types=[
            pltpu.VMEM((EPW,), jnp.int32),
            pltpu.VMEM((EPW, 16), jnp.float32),
            pltpu.SemaphoreType.DMA,
        ],
    )
    def k(table_hbm, idx_hbm, out_hbm, idx_v, rows_v, sem):
        wid = lax.axis_index("s") * 2 + lax.axis_index("c")
        base = wid * EPW
        pltpu.sync_copy(idx_hbm.at[pl.ds(base, EPW)], idx_v)

        def body(c, carry):
            off = pl.multiple_of(c * CH, 8)
            pltpu.async_copy(table_hbm.at[idx_v.at[pl.ds(off, CH)]],
                             rows_v.at[pl.ds(off, CH)], sem)
            return carry

        lax.fori_loop(0, NCH, body, 0)
        pltpu.make_async_copy(table_hbm.at[pl.ds(0, EPW)], rows_v, sem).wait()
        pltpu.sync_copy(rows_v, out_hbm.at[pl.ds(base, EPW)])

    return k(table, idxf)


def _s3_body(g_ref, posp_ref, qpi_ref,
             wc0a_ref, wc0b_ref, bc0_ref, wc1_ref, bc1_ref, wc2_ref, bc2_ref,
             ws0x_ref, ws0p_ref, bs0_ref, ws1_ref, bs1_ref, ws2_ref, bs2_ref,
             q_ref, scene_ref, xs_ref):
    t = pl.program_id(0)
    pos_t = posp_ref[...]
    pj = g_ref[...]
    pd = pj - jnp.broadcast_to(pos_t[:, None, :],
                               (TILE, K, 16)).reshape(TILE * K, 16)
    h1 = _dot(pj, wc0a_ref[...]) + _dot(pd, wc0b_ref[...]) + bc0_ref[...]
    h1 = jnp.maximum(h1, 0.0)
    h2 = jnp.maximum(_dot(h1, wc1_ref[...]) + bc1_ref[...], 0.0)
    msg = _dot(h2, wc2_ref[...])
    x_t = jnp.max(msg.reshape(TILE, K, 256), axis=1) + bc2_ref[...]
    s0 = jnp.maximum(_dot(x_t, ws0x_ref[...]) + _dot(pos_t, ws0p_ref[...])
                     + bs0_ref[...], 0.0)
    s1 = jnp.maximum(_dot(s0, ws1_ref[...]) + bs1_ref[...], 0.0)
    h = _dot(s1, ws2_ref[...]) + bs2_ref[...]
    rowloc = lax.broadcasted_iota(jnp.int32, (TILE, 1), 0) + (t % TPG) * TILE
    hm = jnp.where(rowloc >= NP, jnp.float32(-3e38), h)
    tile_max = jnp.max(hm, axis=0, keepdims=True)

    @pl.when(t % TPG == 0)
    def _():
        scene_ref[...] = jnp.full((1, 1, 1024), -3e38, jnp.float32)

    scene_ref[...] = jnp.maximum(scene_ref[...], tile_max[None])

    @pl.when(t == 0)
    def _():
        q_ref[...] = jnp.zeros((B, 256), jnp.float32)

    xs_ref[...] = x_t
    for b in range(B):
        rb = qpi_ref[0, b] - t * TILE
        @pl.when(jnp.logical_and(rb >= 0, rb < TILE))
        def _():
            q_ref[pl.ds(b, 1), :] = xs_ref[pl.ds(rb, 1), :]


def _s3(g, posp16, qpi, wc0a, wc0b, bc0, wc1, bc1, wc2, bc2,
        ws0x, ws0p, bs0, ws1, bs1, ws2, bs2):
    full = lambda shape: pl.BlockSpec(shape, lambda t: tuple(0 for _ in shape))
    return pl.pallas_call(
        _s3_body,
        grid=(GRID3,),
        scratch_shapes=[pltpu.VMEM((TILE, 256), jnp.float32)],
        in_specs=[
            pl.BlockSpec((TILE * K, 16), lambda t: (t, 0)),
            pl.BlockSpec((TILE, 16), lambda t: (t, 0)),
            pl.BlockSpec(memory_space=pltpu.SMEM),
            full((16, 64)), full((16, 64)), full((1, 64)),
            full((64, 128)), full((1, 128)),
            full((128, 256)), full((1, 256)),
            full((256, 256)), full((16, 256)), full((1, 256)),
            full((256, 512)), full((1, 512)),
            full((512, 1024)), full((1, 1024)),
        ],
        out_specs=[
            pl.BlockSpec((B, 256), lambda t: (0, 0)),
            pl.BlockSpec((1, 1, 1024), lambda t: (t // TPG, 0, 0)),
        ],
        out_shape=[
            jax.ShapeDtypeStruct((B, 256), jnp.float32),
            jax.ShapeDtypeStruct((B, 1, 1024), jnp.float32),
        ],
    )(g, posp16, qpi, wc0a, wc0b, bc0, wc1, bc1, wc2, bc2,
      ws0x, ws0p, bs0, ws1, bs1, ws2, bs2)


def _s4_body(q_ref, sc_ref, wp0a_ref, wp0b_ref, bp0_ref, wp1_ref, bp1_ref,
             wp2_ref, bp2_ref, out_ref):
    e0 = jnp.maximum(_dot(q_ref[...], wp0a_ref[...])
                     + _dot(sc_ref[...], wp0b_ref[...]) + bp0_ref[...], 0.0)
    e1 = jnp.maximum(_dot(e0, wp1_ref[...]) + bp1_ref[...], 0.0)
    out_ref[...] = _dot(e1, wp2_ref[...]) + bp2_ref[...]


def _s4(q, scene, wp0a, wp0b, bp0, wp1, bp1, wp2, bp2):
    return pl.pallas_call(
        _s4_body,
        out_shape=jax.ShapeDtypeStruct((B, 16), jnp.float32),
    )(q, scene, wp0a, wp0b, bp0, wp1, bp1, wp2, bp2)


def kernel(pos, batch, query_point_idx,
           Wc0, bc0, Wc1, bc1, Wc2, bc2,
           Ws0, bs0, Ws1, bs1, Ws2, bs2,
           Wp0, bp0, Wp1, bp1, Wp2, bp2):
    posg = jnp.pad(pos.reshape(B, NP, 3), ((0, 0), (0, NPAD - NP), (0, 1)),
                   constant_values=1e4)
    ptr = jnp.transpose(posg, (0, 2, 1))
    posp16 = jnp.pad(posg.reshape(NT, 4), ((0, 0), (0, 12)))

    idx = _knn(posg, ptr)
    g = _gather_rows(posp16, idx.reshape(E))

    z13 = jnp.zeros((13, 64), jnp.float32)
    wc0a = jnp.concatenate([Wc0[:3], z13])
    wc0b = jnp.concatenate([Wc0[3:], z13])
    ws0x = Ws0[:256]
    ws0p = jnp.concatenate([Ws0[256:], jnp.zeros((13, 256), jnp.float32)])

    qpi = query_point_idx.astype(jnp.int32)
    qpi_pad = ((qpi // NP) * NPAD + qpi % NP).reshape(1, B)

    row = lambda v: v.reshape(1, -1)
    q, scene = _s3(g, posp16, qpi_pad, wc0a, wc0b, row(bc0), Wc1, row(bc1),
                   Wc2, row(bc2), ws0x, ws0p, row(bs0), Ws1, row(bs1),
                   Ws2, row(bs2))
    scene = scene.reshape(B, 1024)
    return _s4(q, scene, Wp0[:256], Wp0[256:], row(bp0), Wp1, row(bp1),
               Wp2, row(bp2))


# speedup vs baseline: 1.1162x; 1.1162x over previous
import functools

import jax
import jax.numpy as jnp
from jax import lax
from jax.experimental import pallas as pl
from jax.experimental.pallas import tpu as pltpu
from jax.experimental.pallas import tpu_sc as plsc

N = 10000
B = 8
NP = N // B
NPAD = 1280
NT = B * NPAD
K = 16
TILE = 640
TPG = NPAD // TILE
GRID3 = NT // TILE
RT = 256
E = NT * K
NWORK = 32
EPW = E // NWORK
CH = 128
NCH = EPW // CH

_PREC = lax.Precision.DEFAULT


def _dot(a, b):
    return lax.dot_general(a, b, (((1,), (0,)), ((), ())),
                           precision=_PREC, preferred_element_type=jnp.float32)


def _knn_body(rows_ref, cols_ref, idx_ref):
    g = pl.program_id(0)
    r = pl.program_id(1)
    rows = rows_ref[0]
    cols = cols_ref[0]
    d = jnp.zeros((RT, NPAD), jnp.float32)
    for c in range(3):
        df = rows[:, c:c + 1] - cols[c:c + 1, :]
        d = d + df * df
    colid = lax.broadcasted_iota(jnp.int32, (RT, NPAD), 1)
    selfid = lax.broadcasted_iota(jnp.int32, (RT, 1), 0) + r * RT
    picks = [selfid]
    d = jnp.where(colid == selfid, jnp.float32(jnp.inf), d)
    for k in range(1, K):
        m = jnp.min(d, axis=1, keepdims=True)
        cand = jnp.where(d == m, colid, jnp.int32(2**30))
        j = jnp.min(cand, axis=1, keepdims=True)
        picks.append(j)
        if k < K - 1:
            d = jnp.where(colid == j, jnp.float32(jnp.inf), d)
    idx_ref[0] = jnp.concatenate(picks, axis=1) + g * NPAD


def _knn(posg, ptr):
    return pl.pallas_call(
        _knn_body,
        grid=(B, NPAD // RT),
        in_specs=[
            pl.BlockSpec((1, RT, 4), lambda g, r: (g, r, 0)),
            pl.BlockSpec((1, 4, NPAD), lambda g, r: (g, 0, 0)),
        ],
        out_specs=pl.BlockSpec((1, RT, K), lambda g, r: (g, r, 0)),
        out_shape=jax.ShapeDtypeStruct((B, NPAD, K), jnp.int32),
    )(posg, ptr)


def _gather_rows(table, idxf):
    mesh = plsc.VectorSubcoreMesh(core_axis_name="c", subcore_axis_name="s")

    @functools.partial(
        pl.kernel, mesh=mesh,
        compiler_params=pltpu.CompilerParams(use_tc_tiling_on_sc=False),
        out_type=jax.ShapeDtypeStruct((E, 16), jnp.float32),
        scratch_types=[
            pltpu.VMEM((EPW,), jnp.int32),
            pltpu.VMEM((EPW, 16), jnp.float32),
            pltpu.SemaphoreType.DMA,
        ],
    )
    def k(table_hbm, idx_hbm, out_hbm, idx_v, rows_v, sem):
        wid = lax.axis_index("s") * 2 + lax.axis_index("c")
        base = wid * EPW
        pltpu.sync_copy(idx_hbm.at[pl.ds(base, EPW)], idx_v)

        def body(c, carry):
            off = pl.multiple_of(c * CH, 8)
            pltpu.async_copy(table_hbm.at[idx_v.at[pl.ds(off, CH)]],
                             rows_v.at[pl.ds(off, CH)], sem)
            return carry

        lax.fori_loop(0, NCH, body, 0)
        pltpu.make_async_copy(table_hbm.at[pl.ds(0, EPW)], rows_v, sem).wait()
        pltpu.sync_copy(rows_v, out_hbm.at[pl.ds(base, EPW)])

    return k(table, idxf)


def _s3_body(g_ref, posp_ref, qpi_ref,
             wc0a_ref, bc0_ref, wc1_ref, bc1_ref, wc2_ref, bc2_ref,
             ws0x_ref, ws0p_ref, bs0_ref, ws1_ref, bs1_ref, ws2_ref, bs2_ref,
             q_ref, scene_ref, xs_ref):
    t = pl.program_id(0)
    pos_t = posp_ref[...]
    pj = g_ref[...]
    pd = pj - jnp.broadcast_to(pos_t[:, None, :],
                               (TILE, K, 16)).reshape(TILE * K, 16)
    feat = jnp.concatenate([pj, pd], axis=1)
    h1 = _dot(feat, wc0a_ref[...]) + bc0_ref[...]
    h1 = jnp.maximum(h1, 0.0)
    h2 = jnp.maximum(_dot(h1, wc1_ref[...]) + bc1_ref[...], 0.0)
    msg = _dot(h2, wc2_ref[...])
    x_t = jnp.max(msg.reshape(TILE, K, 256), axis=1) + bc2_ref[...]
    s0 = jnp.maximum(_dot(x_t, ws0x_ref[...]) + _dot(pos_t, ws0p_ref[...])
                     + bs0_ref[...], 0.0)
    s1 = jnp.maximum(_dot(s0, ws1_ref[...]) + bs1_ref[...], 0.0)
    h = _dot(s1, ws2_ref[...]) + bs2_ref[...]
    rowloc = lax.broadcasted_iota(jnp.int32, (TILE, 1), 0) + (t % TPG) * TILE
    hm = jnp.where(rowloc >= NP, jnp.float32(-3e38), h)
    tile_max = jnp.max(hm, axis=0, keepdims=True)

    @pl.when(t % TPG == 0)
    def _():
        scene_ref[...] = jnp.full((1, 1, 1024), -3e38, jnp.float32)

    scene_ref[...] = jnp.maximum(scene_ref[...], tile_max[None])

    @pl.when(t == 0)
    def _():
        q_ref[...] = jnp.zeros((B, 256), jnp.float32)

    xs_ref[...] = x_t
    for b in range(B):
        rb = qpi_ref[0, b] - t * TILE
        @pl.when(jnp.logical_and(rb >= 0, rb < TILE))
        def _():
            q_ref[pl.ds(b, 1), :] = xs_ref[pl.ds(rb, 1), :]


def _s3(g, posp16, qpi, wc0a, bc0, wc1, bc1, wc2, bc2,
        ws0x, ws0p, bs0, ws1, bs1, ws2, bs2):
    full = lambda shape: pl.BlockSpec(shape, lambda t: tuple(0 for _ in shape))
    return pl.pallas_call(
        _s3_body,
        grid=(GRID3,),
        scratch_shapes=[pltpu.VMEM((TILE, 256), jnp.float32)],
        in_specs=[
            pl.BlockSpec((TILE * K, 16), lambda t: (t, 0)),
            pl.BlockSpec((TILE, 16), lambda t: (t, 0)),
            pl.BlockSpec(memory_space=pltpu.SMEM),
            full((32, 64)), full((1, 64)),
            full((64, 128)), full((1, 128)),
            full((128, 256)), full((1, 256)),
            full((256, 256)), full((16, 256)), full((1, 256)),
            full((256, 512)), full((1, 512)),
            full((512, 1024)), full((1, 1024)),
        ],
        out_specs=[
            pl.BlockSpec((B, 256), lambda t: (0, 0)),
            pl.BlockSpec((1, 1, 1024), lambda t: (t // TPG, 0, 0)),
        ],
        out_shape=[
            jax.ShapeDtypeStruct((B, 256), jnp.float32),
            jax.ShapeDtypeStruct((B, 1, 1024), jnp.float32),
        ],
    )(g, posp16, qpi, wc0a, bc0, wc1, bc1, wc2, bc2,
      ws0x, ws0p, bs0, ws1, bs1, ws2, bs2)


def _s4_body(q_ref, sc_ref, wp0a_ref, wp0b_ref, bp0_ref, wp1_ref, bp1_ref,
             wp2_ref, bp2_ref, out_ref):
    e0 = jnp.maximum(_dot(q_ref[...], wp0a_ref[...])
                     + _dot(sc_ref[...], wp0b_ref[...]) + bp0_ref[...], 0.0)
    e1 = jnp.maximum(_dot(e0, wp1_ref[...]) + bp1_ref[...], 0.0)
    out_ref[...] = _dot(e1, wp2_ref[...]) + bp2_ref[...]


def _s4(q, scene, wp0a, wp0b, bp0, wp1, bp1, wp2, bp2):
    return pl.pallas_call(
        _s4_body,
        out_shape=jax.ShapeDtypeStruct((B, 16), jnp.float32),
    )(q, scene, wp0a, wp0b, bp0, wp1, bp1, wp2, bp2)


def kernel(pos, batch, query_point_idx,
           Wc0, bc0, Wc1, bc1, Wc2, bc2,
           Ws0, bs0, Ws1, bs1, Ws2, bs2,
           Wp0, bp0, Wp1, bp1, Wp2, bp2):
    posg = jnp.pad(pos.reshape(B, NP, 3), ((0, 0), (0, NPAD - NP), (0, 1)),
                   constant_values=1e4)
    ptr = jnp.transpose(posg, (0, 2, 1))
    posp16 = jnp.pad(posg.reshape(NT, 4), ((0, 0), (0, 12)))

    idx = _knn(posg, ptr)
    g = _gather_rows(posp16, idx.reshape(E))

    z13 = jnp.zeros((13, 64), jnp.float32)
    wc0a = jnp.concatenate([Wc0[:3], z13, Wc0[3:], z13])
    ws0x = Ws0[:256]
    ws0p = jnp.concatenate([Ws0[256:], jnp.zeros((13, 256), jnp.float32)])

    qpi = query_point_idx.astype(jnp.int32)
    qpi_pad = ((qpi // NP) * NPAD + qpi % NP).reshape(1, B)

    row = lambda v: v.reshape(1, -1)
    q, scene = _s3(g, posp16, qpi_pad, wc0a, row(bc0), Wc1, row(bc1),
                   Wc2, row(bc2), ws0x, ws0p, row(bs0), Ws1, row(bs1),
                   Ws2, row(bs2))
    scene = scene.reshape(B, 1024)
    return _s4(q, scene, Wp0[:256], Wp0[256:], row(bp0), Wp1, row(bp1),
               Wp2, row(bp2))


# speedup vs baseline: 1.3380x; 1.1986x over previous
import functools

import jax
import jax.numpy as jnp
from jax import lax
from jax.experimental import pallas as pl
from jax.experimental.pallas import tpu as pltpu
from jax.experimental.pallas import tpu_sc as plsc

N = 10000
B = 8
NP = N // B
NPAD = 1280
NT = B * NPAD
K = 16
TILE = 640
TPG = NPAD // TILE
GRID3 = NT // TILE
RT = 256
E = NT * K
NWORK = 32
EPW = E // NWORK
CH = 128
NCH = EPW // CH

_PREC = lax.Precision.DEFAULT


def _dot(a, b):
    return lax.dot_general(a, b, (((1,), (0,)), ((), ())),
                           precision=_PREC, preferred_element_type=jnp.float32)


def _knn_body(rows_ref, cols_ref, idx_ref):
    g = pl.program_id(0)
    r = pl.program_id(1)
    rows = rows_ref[0]
    cols = cols_ref[0]
    d = jnp.zeros((RT, NPAD), jnp.float32)
    for c in range(3):
        df = rows[:, c:c + 1] - cols[c:c + 1, :]
        d = d + df * df
    colid = lax.broadcasted_iota(jnp.int32, (RT, NPAD), 1)
    selfid = lax.broadcasted_iota(jnp.int32, (RT, 1), 0) + r * RT
    picks = [selfid]
    d = jnp.where(colid == selfid, jnp.float32(jnp.inf), d)
    for k in range(1, K):
        j = jnp.argmin(d, axis=1).astype(jnp.int32)[:, None]
        picks.append(j)
        if k < K - 1:
            d = jnp.where(colid == j, jnp.float32(jnp.inf), d)
    idx_ref[0] = jnp.concatenate(picks, axis=1) + g * NPAD


def _knn(posg, ptr):
    return pl.pallas_call(
        _knn_body,
        grid=(B, NPAD // RT),
        in_specs=[
            pl.BlockSpec((1, RT, 4), lambda g, r: (g, r, 0)),
            pl.BlockSpec((1, 4, NPAD), lambda g, r: (g, 0, 0)),
        ],
        out_specs=pl.BlockSpec((1, RT, K), lambda g, r: (g, r, 0)),
        out_shape=jax.ShapeDtypeStruct((B, NPAD, K), jnp.int32),
    )(posg, ptr)


def _gather_rows(table, idxf):
    mesh = plsc.VectorSubcoreMesh(core_axis_name="c", subcore_axis_name="s")

    @functools.partial(
        pl.kernel, mesh=mesh,
        compiler_params=pltpu.CompilerParams(use_tc_tiling_on_sc=False),
        out_type=jax.ShapeDtypeStruct((E, 16), jnp.float32),
        scratch_types=[
            pltpu.VMEM((EPW,), jnp.int32),
            pltpu.VMEM((EPW, 16), jnp.float32),
            pltpu.SemaphoreType.DMA,
        ],
    )
    def k(table_hbm, idx_hbm, out_hbm, idx_v, rows_v, sem):
        wid = lax.axis_index("s") * 2 + lax.axis_index("c")
        base = wid * EPW
        pltpu.sync_copy(idx_hbm.at[pl.ds(base, EPW)], idx_v)

        def body(c, carry):
            off = pl.multiple_of(c * CH, 8)
            pltpu.async_copy(table_hbm.at[idx_v.at[pl.ds(off, CH)]],
                             rows_v.at[pl.ds(off, CH)], sem)
            return carry

        lax.fori_loop(0, NCH, body, 0)
        pltpu.make_async_copy(table_hbm.at[pl.ds(0, EPW)], rows_v, sem).wait()
        pltpu.sync_copy(rows_v, out_hbm.at[pl.ds(base, EPW)])

    return k(table, idxf)


def _s3_body(g_ref, posp_ref, qpi_ref,
             wc0a_ref, bc0_ref, wc1_ref, bc1_ref, wc2_ref, bc2_ref,
             ws0x_ref, ws0p_ref, bs0_ref, ws1_ref, bs1_ref, ws2_ref, bs2_ref,
             q_ref, scene_ref, xs_ref):
    t = pl.program_id(0)
    pos_t = posp_ref[...]
    pj = g_ref[...]
    pd = pj - jnp.broadcast_to(pos_t[:, None, :],
                               (TILE, K, 16)).reshape(TILE * K, 16)
    feat = jnp.concatenate([pj, pd], axis=1)
    h1 = _dot(feat, wc0a_ref[...]) + bc0_ref[...]
    h1 = jnp.maximum(h1, 0.0)
    h2 = jnp.maximum(_dot(h1, wc1_ref[...]) + bc1_ref[...], 0.0)
    msg = _dot(h2, wc2_ref[...])
    x_t = jnp.max(msg.reshape(TILE, K, 256), axis=1) + bc2_ref[...]
    s0 = jnp.maximum(_dot(x_t, ws0x_ref[...]) + _dot(pos_t, ws0p_ref[...])
                     + bs0_ref[...], 0.0)
    s1 = jnp.maximum(_dot(s0, ws1_ref[...]) + bs1_ref[...], 0.0)
    h = _dot(s1, ws2_ref[...]) + bs2_ref[...]
    rowloc = lax.broadcasted_iota(jnp.int32, (TILE, 1), 0) + (t % TPG) * TILE
    hm = jnp.where(rowloc >= NP, jnp.float32(-3e38), h)
    tile_max = jnp.max(hm, axis=0, keepdims=True)

    @pl.when(t % TPG == 0)
    def _():
        scene_ref[...] = jnp.full((1, 1, 1024), -3e38, jnp.float32)

    scene_ref[...] = jnp.maximum(scene_ref[...], tile_max[None])

    @pl.when(t == 0)
    def _():
        q_ref[...] = jnp.zeros((B, 256), jnp.float32)

    xs_ref[...] = x_t
    for b in range(B):
        rb = qpi_ref[0, b] - t * TILE
        @pl.when(jnp.logical_and(rb >= 0, rb < TILE))
        def _():
            q_ref[pl.ds(b, 1), :] = xs_ref[pl.ds(rb, 1), :]


def _s3(g, posp16, qpi, wc0a, bc0, wc1, bc1, wc2, bc2,
        ws0x, ws0p, bs0, ws1, bs1, ws2, bs2):
    full = lambda shape: pl.BlockSpec(shape, lambda t: tuple(0 for _ in shape))
    return pl.pallas_call(
        _s3_body,
        grid=(GRID3,),
        scratch_shapes=[pltpu.VMEM((TILE, 256), jnp.float32)],
        in_specs=[
            pl.BlockSpec((TILE * K, 16), lambda t: (t, 0)),
            pl.BlockSpec((TILE, 16), lambda t: (t, 0)),
            pl.BlockSpec(memory_space=pltpu.SMEM),
            full((32, 64)), full((1, 64)),
            full((64, 128)), full((1, 128)),
            full((128, 256)), full((1, 256)),
            full((256, 256)), full((16, 256)), full((1, 256)),
            full((256, 512)), full((1, 512)),
            full((512, 1024)), full((1, 1024)),
        ],
        out_specs=[
            pl.BlockSpec((B, 256), lambda t: (0, 0)),
            pl.BlockSpec((1, 1, 1024), lambda t: (t // TPG, 0, 0)),
        ],
        out_shape=[
            jax.ShapeDtypeStruct((B, 256), jnp.float32),
            jax.ShapeDtypeStruct((B, 1, 1024), jnp.float32),
        ],
    )(g, posp16, qpi, wc0a, bc0, wc1, bc1, wc2, bc2,
      ws0x, ws0p, bs0, ws1, bs1, ws2, bs2)


def _s4_body(q_ref, sc_ref, wp0a_ref, wp0b_ref, bp0_ref, wp1_ref, bp1_ref,
             wp2_ref, bp2_ref, out_ref):
    e0 = jnp.maximum(_dot(q_ref[...], wp0a_ref[...])
                     + _dot(sc_ref[...], wp0b_ref[...]) + bp0_ref[...], 0.0)
    e1 = jnp.maximum(_dot(e0, wp1_ref[...]) + bp1_ref[...], 0.0)
    out_ref[...] = _dot(e1, wp2_ref[...]) + bp2_ref[...]


def _s4(q, scene, wp0a, wp0b, bp0, wp1, bp1, wp2, bp2):
    return pl.pallas_call(
        _s4_body,
        out_shape=jax.ShapeDtypeStruct((B, 16), jnp.float32),
    )(q, scene, wp0a, wp0b, bp0, wp1, bp1, wp2, bp2)


def kernel(pos, batch, query_point_idx,
           Wc0, bc0, Wc1, bc1, Wc2, bc2,
           Ws0, bs0, Ws1, bs1, Ws2, bs2,
           Wp0, bp0, Wp1, bp1, Wp2, bp2):
    posg = jnp.pad(pos.reshape(B, NP, 3), ((0, 0), (0, NPAD - NP), (0, 1)),
                   constant_values=1e4)
    ptr = jnp.transpose(posg, (0, 2, 1))
    posp16 = jnp.pad(posg.reshape(NT, 4), ((0, 0), (0, 12)))

    idx = _knn(posg, ptr)
    g = _gather_rows(posp16, idx.reshape(E))

    z13 = jnp.zeros((13, 64), jnp.float32)
    wc0a = jnp.concatenate([Wc0[:3], z13, Wc0[3:], z13])
    ws0x = Ws0[:256]
    ws0p = jnp.concatenate([Ws0[256:], jnp.zeros((13, 256), jnp.float32)])

    qpi = query_point_idx.astype(jnp.int32)
    qpi_pad = ((qpi // NP) * NPAD + qpi % NP).reshape(1, B)

    row = lambda v: v.reshape(1, -1)
    q, scene = _s3(g, posp16, qpi_pad, wc0a, row(bc0), Wc1, row(bc1),
                   Wc2, row(bc2), ws0x, ws0p, row(bs0), Ws1, row(bs1),
                   Ws2, row(bs2))
    scene = scene.reshape(B, 1024)
    return _s4(q, scene, Wp0[:256], Wp0[256:], row(bp0), Wp1, row(bp1),
               Wp2, row(bp2))


# speedup vs baseline: 1.3483x; 1.0077x over previous
import functools

import jax
import jax.numpy as jnp
from jax import lax
from jax.experimental import pallas as pl
from jax.experimental.pallas import tpu as pltpu
from jax.experimental.pallas import tpu_sc as plsc

N = 10000
B = 8
NP = N // B
NPAD = 1280
NT = B * NPAD
K = 16
TILE = 1280
TPG = NPAD // TILE
GRID3 = NT // TILE
RT = 256
E = NT * K
NWORK = 32
EPW = E // NWORK
CH = 128
NCH = EPW // CH

_PREC = lax.Precision.DEFAULT


def _dot(a, b):
    return lax.dot_general(a, b, (((1,), (0,)), ((), ())),
                           precision=_PREC, preferred_element_type=jnp.float32)


def _knn_body(rows_ref, cols_ref, idx_ref):
    g = pl.program_id(0)
    r = pl.program_id(1)
    rows = rows_ref[0]
    cols = cols_ref[0]
    d = jnp.zeros((RT, NPAD), jnp.float32)
    for c in range(3):
        df = rows[:, c:c + 1] - cols[c:c + 1, :]
        d = d + df * df
    colid = lax.broadcasted_iota(jnp.int32, (RT, NPAD), 1)
    selfid = lax.broadcasted_iota(jnp.int32, (RT, 1), 0) + r * RT
    picks = [selfid]
    d = jnp.where(colid == selfid, jnp.float32(jnp.inf), d)
    for k in range(1, K):
        j = jnp.argmin(d, axis=1).astype(jnp.int32)[:, None]
        picks.append(j)
        if k < K - 1:
            d = jnp.where(colid == j, jnp.float32(jnp.inf), d)
    idx_ref[0] = jnp.concatenate(picks, axis=1) + g * NPAD


def _knn(posg, ptr):
    return pl.pallas_call(
        _knn_body,
        grid=(B, NPAD // RT),
        in_specs=[
            pl.BlockSpec((1, RT, 4), lambda g, r: (g, r, 0)),
            pl.BlockSpec((1, 4, NPAD), lambda g, r: (g, 0, 0)),
        ],
        out_specs=pl.BlockSpec((1, RT, K), lambda g, r: (g, r, 0)),
        out_shape=jax.ShapeDtypeStruct((B, NPAD, K), jnp.int32),
    )(posg, ptr)


def _gather_rows(table, idxf):
    mesh = plsc.VectorSubcoreMesh(core_axis_name="c", subcore_axis_name="s")

    @functools.partial(
        pl.kernel, mesh=mesh,
        compiler_params=pltpu.CompilerParams(use_tc_tiling_on_sc=False),
        out_type=jax.ShapeDtypeStruct((E, 16), jnp.float32),
        scratch_types=[
            pltpu.VMEM((EPW,), jnp.int32),
            pltpu.VMEM((EPW, 16), jnp.float32),
            pltpu.SemaphoreType.DMA,
        ],
    )
    def k(table_hbm, idx_hbm, out_hbm, idx_v, rows_v, sem):
        wid = lax.axis_index("s") * 2 + lax.axis_index("c")
        base = wid * EPW
        pltpu.sync_copy(idx_hbm.at[pl.ds(base, EPW)], idx_v)

        def body(c, carry):
            off = pl.multiple_of(c * CH, 8)
            pltpu.async_copy(table_hbm.at[idx_v.at[pl.ds(off, CH)]],
                             rows_v.at[pl.ds(off, CH)], sem)
            return carry

        lax.fori_loop(0, NCH, body, 0)
        pltpu.make_async_copy(table_hbm.at[pl.ds(0, EPW)], rows_v, sem).wait()
        pltpu.sync_copy(rows_v, out_hbm.at[pl.ds(base, EPW)])

    return k(table, idxf)


def _s3_body(g_ref, posp_ref, qpi_ref,
             wc0a_ref, bc0_ref, wc1_ref, bc1_ref, wc2_ref, bc2_ref,
             ws0x_ref, ws0p_ref, bs0_ref, ws1_ref, bs1_ref, ws2_ref, bs2_ref,
             q_ref, scene_ref, xs_ref):
    t = pl.program_id(0)
    pos_t = posp_ref[...]
    pj = g_ref[...]
    pd = pj - jnp.broadcast_to(pos_t[:, None, :],
                               (TILE, K, 16)).reshape(TILE * K, 16)
    feat = jnp.concatenate([pj, pd], axis=1)
    h1 = _dot(feat, wc0a_ref[...]) + bc0_ref[...]
    h1 = jnp.maximum(h1, 0.0)
    h2 = jnp.maximum(_dot(h1, wc1_ref[...]) + bc1_ref[...], 0.0)
    msg = _dot(h2, wc2_ref[...])
    x_t = jnp.max(msg.reshape(TILE, K, 256), axis=1) + bc2_ref[...]
    s0 = jnp.maximum(_dot(x_t, ws0x_ref[...]) + _dot(pos_t, ws0p_ref[...])
                     + bs0_ref[...], 0.0)
    s1 = jnp.maximum(_dot(s0, ws1_ref[...]) + bs1_ref[...], 0.0)
    h = _dot(s1, ws2_ref[...]) + bs2_ref[...]
    rowloc = lax.broadcasted_iota(jnp.int32, (TILE, 1), 0) + (t % TPG) * TILE
    hm = jnp.where(rowloc >= NP, jnp.float32(-3e38), h)
    tile_max = jnp.max(hm, axis=0, keepdims=True)

    @pl.when(t % TPG == 0)
    def _():
        scene_ref[...] = jnp.full((1, 1, 1024), -3e38, jnp.float32)

    scene_ref[...] = jnp.maximum(scene_ref[...], tile_max[None])

    @pl.when(t == 0)
    def _():
        q_ref[...] = jnp.zeros((B, 256), jnp.float32)

    xs_ref[...] = x_t
    for b in range(B):
        rb = qpi_ref[0, b] - t * TILE
        @pl.when(jnp.logical_and(rb >= 0, rb < TILE))
        def _():
            q_ref[pl.ds(b, 1), :] = xs_ref[pl.ds(rb, 1), :]


def _s3(g, posp16, qpi, wc0a, bc0, wc1, bc1, wc2, bc2,
        ws0x, ws0p, bs0, ws1, bs1, ws2, bs2):
    full = lambda shape: pl.BlockSpec(shape, lambda t: tuple(0 for _ in shape))
    return pl.pallas_call(
        _s3_body,
        grid=(GRID3,),
        scratch_shapes=[pltpu.VMEM((TILE, 256), jnp.float32)],
        in_specs=[
            pl.BlockSpec((TILE * K, 16), lambda t: (t, 0)),
            pl.BlockSpec((TILE, 16), lambda t: (t, 0)),
            pl.BlockSpec(memory_space=pltpu.SMEM),
            full((32, 64)), full((1, 64)),
            full((64, 128)), full((1, 128)),
            full((128, 256)), full((1, 256)),
            full((256, 256)), full((16, 256)), full((1, 256)),
            full((256, 512)), full((1, 512)),
            full((512, 1024)), full((1, 1024)),
        ],
        out_specs=[
            pl.BlockSpec((B, 256), lambda t: (0, 0)),
            pl.BlockSpec((1, 1, 1024), lambda t: (t // TPG, 0, 0)),
        ],
        out_shape=[
            jax.ShapeDtypeStruct((B, 256), jnp.float32),
            jax.ShapeDtypeStruct((B, 1, 1024), jnp.float32),
        ],
    )(g, posp16, qpi, wc0a, bc0, wc1, bc1, wc2, bc2,
      ws0x, ws0p, bs0, ws1, bs1, ws2, bs2)


def _s4_body(q_ref, sc_ref, wp0a_ref, wp0b_ref, bp0_ref, wp1_ref, bp1_ref,
             wp2_ref, bp2_ref, out_ref):
    e0 = jnp.maximum(_dot(q_ref[...], wp0a_ref[...])
                     + _dot(sc_ref[...], wp0b_ref[...]) + bp0_ref[...], 0.0)
    e1 = jnp.maximum(_dot(e0, wp1_ref[...]) + bp1_ref[...], 0.0)
    out_ref[...] = _dot(e1, wp2_ref[...]) + bp2_ref[...]


def _s4(q, scene, wp0a, wp0b, bp0, wp1, bp1, wp2, bp2):
    return pl.pallas_call(
        _s4_body,
        out_shape=jax.ShapeDtypeStruct((B, 16), jnp.float32),
    )(q, scene, wp0a, wp0b, bp0, wp1, bp1, wp2, bp2)


def kernel(pos, batch, query_point_idx,
           Wc0, bc0, Wc1, bc1, Wc2, bc2,
           Ws0, bs0, Ws1, bs1, Ws2, bs2,
           Wp0, bp0, Wp1, bp1, Wp2, bp2):
    posg = jnp.pad(pos.reshape(B, NP, 3), ((0, 0), (0, NPAD - NP), (0, 1)),
                   constant_values=1e4)
    ptr = jnp.transpose(posg, (0, 2, 1))
    posp16 = jnp.pad(posg.reshape(NT, 4), ((0, 0), (0, 12)))

    idx = _knn(posg, ptr)
    g = _gather_rows(posp16, idx.reshape(E))

    z13 = jnp.zeros((13, 64), jnp.float32)
    wc0a = jnp.concatenate([Wc0[:3], z13, Wc0[3:], z13])
    ws0x = Ws0[:256]
    ws0p = jnp.concatenate([Ws0[256:], jnp.zeros((13, 256), jnp.float32)])

    qpi = query_point_idx.astype(jnp.int32)
    qpi_pad = ((qpi // NP) * NPAD + qpi % NP).reshape(1, B)

    row = lambda v: v.reshape(1, -1)
    q, scene = _s3(g, posp16, qpi_pad, wc0a, row(bc0), Wc1, row(bc1),
                   Wc2, row(bc2), ws0x, ws0p, row(bs0), Ws1, row(bs1),
                   Ws2, row(bs2))
    scene = scene.reshape(B, 1024)
    return _s4(q, scene, Wp0[:256], Wp0[256:], row(bp0), Wp1, row(bp1),
               Wp2, row(bp2))
